# Initial kernel scaffold; baseline (speedup 1.0000x reference)
import jax
import jax.numpy as jnp
from jax.experimental import pallas as pl


def kernel(x, table, pe):
    raise NotImplementedError("write your pallas kernel here")


# speedup vs baseline: 1.3061x; 1.3061x over previous
import functools

import jax
import jax.numpy as jnp
from jax import lax
from jax.experimental import pallas as pl
from jax.experimental.pallas import tpu as pltpu
from jax.experimental.pallas import tpu_sc as plsc

S = 2048
B = 4
D = 128
N = S * B
NW = 32
RPW = N // NW
SPW = S // NW
LANES = 16


def _emb_body(x_hbm, pe_hbm, table_hbm, out_hbm, idx_v, rows_v, pe_v, sem):
    wid = lax.axis_index("s") * 2 + lax.axis_index("c")
    base = wid * RPW
    s0 = wid * SPW

    pltpu.sync_copy(x_hbm.at[pl.ds(wid * (RPW // 128), RPW // 128)], idx_v)
    pltpu.sync_copy(pe_hbm.at[pl.ds(s0, SPW)], pe_v)

    cps = []
    for c in range(RPW // 128):
        cps.append(
            pltpu.async_copy(
                table_hbm.at[idx_v.at[c]],
                rows_v.at[pl.ds(c * 128, 128)],
                sem,
            )
        )
    for cp in cps:
        cp.wait()

    def add_body(r, _):
        row = r * B
        for j in range(D // LANES):
            sl = pl.ds(j * LANES, LANES)
            p = pe_v[r, sl]
            for b in range(B):
                rows_v[row + b, sl] = rows_v[row + b, sl] + p
        return _

    lax.fori_loop(0, SPW, add_body, None)

    pltpu.sync_copy(rows_v, out_hbm.at[pl.ds(base, RPW)])


@jax.jit
def _emb(x2, pe2, table):
    mesh = plsc.VectorSubcoreMesh(core_axis_name="c", subcore_axis_name="s")
    f = functools.partial(
        pl.kernel,
        mesh=mesh,
        out_type=jax.ShapeDtypeStruct((N, D), jnp.float32),
        scratch_types=[
            pltpu.VMEM((RPW // 128, 128), jnp.int32),
            pltpu.VMEM((RPW, D), jnp.float32),
            pltpu.VMEM((SPW, D), jnp.float32),
            pltpu.SemaphoreType.DMA,
        ],
    )(_emb_body)
    return f(x2, pe2, table)


def kernel(x, table, pe):
    x2 = x.reshape(N // 128, 128)
    pe2 = pe.reshape(pe.shape[0], D)[:S]
    out = _emb(x2, pe2, table)
    return out.reshape(S, B, D)


# speedup vs baseline: 1.3588x; 1.0404x over previous
import functools

import jax
import jax.numpy as jnp
from jax import lax
from jax.experimental import pallas as pl
from jax.experimental.pallas import tpu as pltpu
from jax.experimental.pallas import tpu_sc as plsc

S = 2048
B = 4
D = 128
N = S * B
NW = 32
RPW = N // NW
SPW = S // NW
LANES = 16


def _emb_body(x_hbm, pe_hbm, table_hbm, out_hbm, idx_v, rows_v, pe_v,
              sem_g0, sem_g1, sem_o0, sem_o1):
    wid = lax.axis_index("s") * 2 + lax.axis_index("c")
    base = wid * RPW
    s0 = wid * SPW

    pltpu.sync_copy(x_hbm.at[pl.ds(wid * (RPW // 128), RPW // 128)], idx_v)

    g0 = pltpu.async_copy(table_hbm.at[idx_v.at[0]],
                          rows_v.at[pl.ds(0, 128)], sem_g0)
    g1 = pltpu.async_copy(table_hbm.at[idx_v.at[1]],
                          rows_v.at[pl.ds(128, 128)], sem_g1)
    pltpu.sync_copy(pe_hbm.at[pl.ds(s0, SPW)], pe_v)

    def add_body(r, _):
        row = r * B
        for j in range(D // LANES):
            sl = pl.ds(j * LANES, LANES)
            p = pe_v[r, sl]
            for b in range(B):
                rows_v[row + b, sl] = rows_v[row + b, sl] + p
        return _

    g0.wait()
    lax.fori_loop(0, SPW // 2, add_body, None)
    o0 = pltpu.async_copy(rows_v.at[pl.ds(0, 128)],
                          out_hbm.at[pl.ds(base, 128)], sem_o0)
    g1.wait()
    lax.fori_loop(SPW // 2, SPW, add_body, None)
    o1 = pltpu.async_copy(rows_v.at[pl.ds(128, 128)],
                          out_hbm.at[pl.ds(base + 128, 128)], sem_o1)
    o0.wait()
    o1.wait()


@jax.jit
def _emb(x2, pe2, table):
    mesh = plsc.VectorSubcoreMesh(core_axis_name="c", subcore_axis_name="s")
    f = functools.partial(
        pl.kernel,
        mesh=mesh,
        out_type=jax.ShapeDtypeStruct((N, D), jnp.float32),
        scratch_types=[
            pltpu.VMEM((RPW // 128, 128), jnp.int32),
            pltpu.VMEM((RPW, D), jnp.float32),
            pltpu.VMEM((SPW, D), jnp.float32),
            pltpu.SemaphoreType.DMA,
            pltpu.SemaphoreType.DMA,
            pltpu.SemaphoreType.DMA,
            pltpu.SemaphoreType.DMA,
        ],
    )(_emb_body)
    return f(x2, pe2, table)


def kernel(x, table, pe):
    x2 = x.reshape(N // 128, 128)
    pe2 = pe.reshape(pe.shape[0], D)[:S]
    out = _emb(x2, pe2, table)
    return out.reshape(S, B, D)
